# initial kernel scaffold (unmeasured)
import jax
import jax.numpy as jnp
from jax import lax
from jax.experimental import pallas as pl
from jax.experimental.pallas import tpu as pltpu


def kernel(
    x,
):
    def body(*refs):
        pass

    out_shape = jax.ShapeDtypeStruct(..., jnp.float32)
    return pl.pallas_call(body, out_shape=out_shape)(...)


# baseline (device time: 86948 ns/iter reference)
import jax
import jax.numpy as jnp
from jax import lax
from jax.experimental import pallas as pl
from jax.experimental.pallas import tpu as pltpu

N_CHUNKS = 8


def kernel(x):
    m, n = x.shape
    half_m = m // 2
    n_out = n // 2
    chunk = half_m // N_CHUNKS
    out_dtype = jnp.bfloat16

    def body(x_ref, out_ref, send_buf, y_buf, x_buf,
             y_send_sems, y_recv_sems, x_send_sems, x_recv_sems):
        my_x = lax.axis_index("x")
        my_y = lax.axis_index("y")
        other_x = 1 - my_x
        other_y = 1 - my_y

        barrier_sem = pltpu.get_barrier_semaphore()
        pl.semaphore_signal(barrier_sem, inc=1, device_id=(my_x, other_y),
                            device_id_type=pl.DeviceIdType.MESH)
        pl.semaphore_signal(barrier_sem, inc=1, device_id=(other_x, my_y),
                            device_id_type=pl.DeviceIdType.MESH)
        pl.semaphore_wait(barrier_sem, 2)

        send_buf[...] = x_ref[pl.ds(my_x * half_m, half_m),
                              pl.ds(other_y * n_out, n_out)].astype(out_dtype)

        y_rdmas = []
        for c in range(N_CHUNKS):
            r = pltpu.make_async_remote_copy(
                src_ref=send_buf.at[pl.ds(c * chunk, chunk), :],
                dst_ref=y_buf.at[pl.ds(c * chunk, chunk), :],
                send_sem=y_send_sems.at[c],
                recv_sem=y_recv_sems.at[c],
                device_id=(my_x, other_y),
                device_id_type=pl.DeviceIdType.MESH,
            )
            r.start()
            y_rdmas.append(r)

        out_ref[pl.ds(my_y * m, m), :] = (
            x_ref[:, pl.ds(my_y * n_out, n_out)].astype(out_dtype))

        x_rdmas = []
        for c in range(N_CHUNKS):
            y_rdmas[c].wait_recv()
            r = pltpu.make_async_remote_copy(
                src_ref=y_buf.at[pl.ds(c * chunk, chunk), :],
                dst_ref=x_buf.at[pl.ds(c * chunk, chunk), :],
                send_sem=x_send_sems.at[c],
                recv_sem=x_recv_sems.at[c],
                device_id=(other_x, my_y),
                device_id_type=pl.DeviceIdType.MESH,
            )
            r.start()
            x_rdmas.append(r)
            out_ref[pl.ds(other_y * m + my_x * half_m + c * chunk, chunk), :] = (
                y_buf[pl.ds(c * chunk, chunk), :])

        for c in range(N_CHUNKS):
            x_rdmas[c].wait_recv()
            out_ref[pl.ds(other_y * m + other_x * half_m + c * chunk, chunk), :] = (
                x_buf[pl.ds(c * chunk, chunk), :])

        for c in range(N_CHUNKS):
            y_rdmas[c].wait_send()
            x_rdmas[c].wait_send()

    return pl.pallas_call(
        body,
        out_shape=jax.ShapeDtypeStruct((2 * m, n_out), out_dtype),
        in_specs=[pl.BlockSpec(memory_space=pltpu.VMEM)],
        out_specs=pl.BlockSpec(memory_space=pltpu.VMEM),
        scratch_shapes=[
            pltpu.VMEM((half_m, n_out), out_dtype),
            pltpu.VMEM((half_m, n_out), out_dtype),
            pltpu.VMEM((half_m, n_out), out_dtype),
            pltpu.SemaphoreType.DMA((N_CHUNKS,)),
            pltpu.SemaphoreType.DMA((N_CHUNKS,)),
            pltpu.SemaphoreType.DMA((N_CHUNKS,)),
            pltpu.SemaphoreType.DMA((N_CHUNKS,)),
        ],
        compiler_params=pltpu.CompilerParams(
            collective_id=0,
            vmem_limit_bytes=100 * 1024 * 1024,
        ),
    )(x)


# device time: 72595 ns/iter; 1.1977x vs baseline; 1.1977x over previous
import jax
import jax.numpy as jnp
from jax import lax
from jax.experimental import pallas as pl
from jax.experimental.pallas import tpu as pltpu

N_CHUNKS = 8
N_KEEP = 8


def kernel(x):
    m, n = x.shape
    half_m = m // 2
    n_out = n // 2
    chunk = half_m // N_CHUNKS
    kchunk = m // N_KEEP
    out_dtype = jnp.bfloat16

    def body(x_hbm, out_hbm, send_buf, y_buf, x_buf, s_stage, k_stage, k_buf,
             s_in_sems, k_in_sems, k_out_sems, out_sem,
             y_send_sems, y_recv_sems, x_send_sems, x_recv_sems):
        my_x = lax.axis_index("x")
        my_y = lax.axis_index("y")
        other_x = 1 - my_x
        other_y = 1 - my_y

        barrier_sem = pltpu.get_barrier_semaphore()
        pl.semaphore_signal(barrier_sem, inc=1, device_id=(my_x, other_y),
                            device_id_type=pl.DeviceIdType.MESH)
        pl.semaphore_signal(barrier_sem, inc=1, device_id=(other_x, my_y),
                            device_id_type=pl.DeviceIdType.MESH)
        pl.semaphore_wait(barrier_sem, 2)

        send_row0 = my_x * half_m
        col_send = other_y * n_out
        col_keep = my_y * n_out
        yrecv_rows = other_y * m + my_x * half_m
        xrecv_rows = other_y * m + other_x * half_m

        def s_load(c, slot):
            return pltpu.make_async_copy(
                x_hbm.at[pl.ds(send_row0 + c * chunk, chunk),
                         pl.ds(col_send, n_out)],
                s_stage.at[slot], s_in_sems.at[slot])

        def k_load(k, slot):
            return pltpu.make_async_copy(
                x_hbm.at[pl.ds(k * kchunk, kchunk), pl.ds(col_keep, n_out)],
                k_stage.at[slot], k_in_sems.at[slot])

        def k_store(k, slot):
            return pltpu.make_async_copy(
                k_buf.at[slot],
                out_hbm.at[pl.ds(my_y * m + k * kchunk, kchunk), :],
                k_out_sems.at[slot])

        s_load(0, 0).start()
        s_load(1, 1).start()
        k_load(0, 0).start()
        k_load(1, 1).start()

        y_rdmas = []
        for c in range(N_CHUNKS):
            slot = c % 2
            s_load(c, slot).wait()
            send_buf[pl.ds(c * chunk, chunk), :] = (
                s_stage[slot, :, :].astype(out_dtype))
            r = pltpu.make_async_remote_copy(
                src_ref=send_buf.at[pl.ds(c * chunk, chunk), :],
                dst_ref=y_buf.at[pl.ds(c * chunk, chunk), :],
                send_sem=y_send_sems.at[c],
                recv_sem=y_recv_sems.at[c],
                device_id=(my_x, other_y),
                device_id_type=pl.DeviceIdType.MESH,
            )
            r.start()
            y_rdmas.append(r)
            if c + 2 < N_CHUNKS:
                s_load(c + 2, slot).start()

        x_rdmas = []
        out_copies = []
        for c in range(N_CHUNKS):
            y_rdmas[c].wait_recv()
            r = pltpu.make_async_remote_copy(
                src_ref=y_buf.at[pl.ds(c * chunk, chunk), :],
                dst_ref=x_buf.at[pl.ds(c * chunk, chunk), :],
                send_sem=x_send_sems.at[c],
                recv_sem=x_recv_sems.at[c],
                device_id=(other_x, my_y),
                device_id_type=pl.DeviceIdType.MESH,
            )
            r.start()
            x_rdmas.append(r)
            cp = pltpu.make_async_copy(
                y_buf.at[pl.ds(c * chunk, chunk), :],
                out_hbm.at[pl.ds(yrecv_rows + c * chunk, chunk), :],
                out_sem)
            cp.start()
            out_copies.append(cp)

            k = c
            slot = k % 2
            k_load(k, slot).wait()
            if k >= 2:
                k_store(k - 2, slot).wait()
            k_buf[slot, :, :] = k_stage[slot, :, :].astype(out_dtype)
            k_store(k, slot).start()
            if k + 2 < N_KEEP:
                k_load(k + 2, slot).start()

        for c in range(N_CHUNKS):
            x_rdmas[c].wait_recv()
            cp = pltpu.make_async_copy(
                x_buf.at[pl.ds(c * chunk, chunk), :],
                out_hbm.at[pl.ds(xrecv_rows + c * chunk, chunk), :],
                out_sem)
            cp.start()
            out_copies.append(cp)

        for cp in out_copies:
            cp.wait()
        for k in (N_KEEP - 2, N_KEEP - 1):
            k_store(k, k % 2).wait()
        for r in y_rdmas:
            r.wait_send()
        for r in x_rdmas:
            r.wait_send()

    return pl.pallas_call(
        body,
        out_shape=jax.ShapeDtypeStruct((2 * m, n_out), out_dtype),
        in_specs=[pl.BlockSpec(memory_space=pl.ANY)],
        out_specs=pl.BlockSpec(memory_space=pl.ANY),
        scratch_shapes=[
            pltpu.VMEM((half_m, n_out), out_dtype),
            pltpu.VMEM((half_m, n_out), out_dtype),
            pltpu.VMEM((half_m, n_out), out_dtype),
            pltpu.VMEM((2, chunk, n_out), jnp.float32),
            pltpu.VMEM((2, kchunk, n_out), jnp.float32),
            pltpu.VMEM((2, kchunk, n_out), out_dtype),
            pltpu.SemaphoreType.DMA((2,)),
            pltpu.SemaphoreType.DMA((2,)),
            pltpu.SemaphoreType.DMA((2,)),
            pltpu.SemaphoreType.DMA,
            pltpu.SemaphoreType.DMA((N_CHUNKS,)),
            pltpu.SemaphoreType.DMA((N_CHUNKS,)),
            pltpu.SemaphoreType.DMA((N_CHUNKS,)),
            pltpu.SemaphoreType.DMA((N_CHUNKS,)),
        ],
        compiler_params=pltpu.CompilerParams(
            collective_id=0,
            vmem_limit_bytes=100 * 1024 * 1024,
        ),
    )(x)


# device time: 72204 ns/iter; 1.2042x vs baseline; 1.0054x over previous
import os

import jax
import jax.numpy as jnp
from jax import lax
from jax.experimental import pallas as pl
from jax.experimental.pallas import tpu as pltpu

N_CHUNKS = 8
N_KEEP = 8

_SKIP_KEEP = os.environ.get("SKIP_KEEP") == "1"
_SKIP_WIRE = os.environ.get("SKIP_WIRE") == "1"
_SKIP_X = os.environ.get("SKIP_X") == "1"


def kernel(x):
    m, n = x.shape
    half_m = m // 2
    n_out = n // 2
    chunk = half_m // N_CHUNKS
    kchunk = m // N_KEEP
    out_dtype = jnp.bfloat16

    def body(x_hbm, out_hbm, send_buf, s_stage, k_stage, k_buf,
             s_in_sems, k_in_sems, k_out_sems,
             y_send_sem, y_recv_sems, x_send_sem, x_recv_sem):
        my_x = lax.axis_index("x")
        my_y = lax.axis_index("y")
        other_x = 1 - my_x
        other_y = 1 - my_y

        send_row0 = my_x * half_m
        col_send = other_y * n_out
        col_keep = my_y * n_out
        sent_rows = my_y * m + send_row0
        yrecv_rows = other_y * m + my_x * half_m
        xrecv_rows = other_y * m + other_x * half_m

        def s_load(c, slot):
            return pltpu.make_async_copy(
                x_hbm.at[pl.ds(send_row0 + c * chunk, chunk),
                         pl.ds(col_send, n_out)],
                s_stage.at[slot], s_in_sems.at[slot])

        def k_load(k, slot):
            return pltpu.make_async_copy(
                x_hbm.at[pl.ds(k * kchunk, kchunk), pl.ds(col_keep, n_out)],
                k_stage.at[slot], k_in_sems.at[slot])

        def k_store(k, slot):
            return pltpu.make_async_copy(
                k_buf.at[slot],
                out_hbm.at[pl.ds(my_y * m + k * kchunk, kchunk), :],
                k_out_sems.at[slot])

        s_load(0, 0).start()
        s_load(1, 1).start()
        if not _SKIP_KEEP:
            k_load(0, 0).start()
            k_load(1, 1).start()

        barrier_sem = pltpu.get_barrier_semaphore()
        pl.semaphore_signal(barrier_sem, inc=1, device_id=(my_x, other_y),
                            device_id_type=pl.DeviceIdType.MESH)
        pl.semaphore_signal(barrier_sem, inc=1, device_id=(other_x, my_y),
                            device_id_type=pl.DeviceIdType.MESH)
        pl.semaphore_wait(barrier_sem, 2)

        y_rdmas = []
        for c in range(N_CHUNKS):
            slot = c % 2
            s_load(c, slot).wait()
            send_buf[pl.ds(c * chunk, chunk), :] = (
                s_stage[slot, :, :].astype(out_dtype))
            if not _SKIP_WIRE:
                r = pltpu.make_async_remote_copy(
                    src_ref=send_buf.at[pl.ds(c * chunk, chunk), :],
                    dst_ref=out_hbm.at[pl.ds(sent_rows + c * chunk, chunk), :],
                    send_sem=y_send_sem,
                    recv_sem=y_recv_sems.at[c],
                    device_id=(my_x, other_y),
                    device_id_type=pl.DeviceIdType.MESH,
                )
                r.start()
                y_rdmas.append(r)
            if c + 2 < N_CHUNKS:
                s_load(c + 2, slot).start()

        for c in range(N_CHUNKS):
            if not _SKIP_WIRE:
                y_rdmas[c].wait_recv()
            if not _SKIP_WIRE and not _SKIP_X:
                r = pltpu.make_async_remote_copy(
                    src_ref=out_hbm.at[pl.ds(yrecv_rows + c * chunk, chunk), :],
                    dst_ref=out_hbm.at[pl.ds(yrecv_rows + c * chunk, chunk), :],
                    send_sem=x_send_sem,
                    recv_sem=x_recv_sem,
                    device_id=(other_x, my_y),
                    device_id_type=pl.DeviceIdType.MESH,
                )
                r.start()

            if not _SKIP_KEEP and c % (N_CHUNKS // N_KEEP) == 0:
                k = c // (N_CHUNKS // N_KEEP)
                slot = k % 2
                k_load(k, slot).wait()
                if k >= 2:
                    k_store(k - 2, slot).wait()
                k_buf[slot, :, :] = k_stage[slot, :, :].astype(out_dtype)
                k_store(k, slot).start()
                if k + 2 < N_KEEP:
                    k_load(k + 2, slot).start()

        if not _SKIP_WIRE and not _SKIP_X:
            whole_x = pltpu.make_async_remote_copy(
                src_ref=out_hbm.at[pl.ds(yrecv_rows, half_m), :],
                dst_ref=out_hbm.at[pl.ds(xrecv_rows, half_m), :],
                send_sem=x_send_sem,
                recv_sem=x_recv_sem,
                device_id=(other_x, my_y),
                device_id_type=pl.DeviceIdType.MESH,
            )
            whole_x.wait_recv()
            whole_x.wait_send()
        if not _SKIP_WIRE:
            whole_y = pltpu.make_async_remote_copy(
                src_ref=send_buf.at[...],
                dst_ref=out_hbm.at[pl.ds(sent_rows, half_m), :],
                send_sem=y_send_sem,
                recv_sem=y_recv_sems.at[0],
                device_id=(my_x, other_y),
                device_id_type=pl.DeviceIdType.MESH,
            )
            whole_y.wait_send()
        if not _SKIP_KEEP:
            for k in (N_KEEP - 2, N_KEEP - 1):
                k_store(k, k % 2).wait()

    return pl.pallas_call(
        body,
        out_shape=jax.ShapeDtypeStruct((2 * m, n_out), out_dtype),
        in_specs=[pl.BlockSpec(memory_space=pl.ANY)],
        out_specs=pl.BlockSpec(memory_space=pl.ANY),
        scratch_shapes=[
            pltpu.VMEM((half_m, n_out), out_dtype),
            pltpu.VMEM((2, chunk, n_out), jnp.float32),
            pltpu.VMEM((2, kchunk, n_out), jnp.float32),
            pltpu.VMEM((2, kchunk, n_out), out_dtype),
            pltpu.SemaphoreType.DMA((2,)),
            pltpu.SemaphoreType.DMA((2,)),
            pltpu.SemaphoreType.DMA((2,)),
            pltpu.SemaphoreType.DMA,
            pltpu.SemaphoreType.DMA((N_CHUNKS,)),
            pltpu.SemaphoreType.DMA,
            pltpu.SemaphoreType.DMA,
        ],
        compiler_params=pltpu.CompilerParams(
            collective_id=0,
            vmem_limit_bytes=100 * 1024 * 1024,
        ),
    )(x)
